# Initial kernel scaffold; baseline (speedup 1.0000x reference)
import jax
import jax.numpy as jnp
from jax.experimental import pallas as pl


def kernel(news_title, news_topic, news_subtopic, title_vectors, topic_table, subtopic_table, W, b):
    raise NotImplementedError("write your pallas kernel here")


# speedup vs baseline: 1.7050x; 1.7050x over previous
import functools

import jax
import jax.numpy as jnp
from jax import lax
from jax.experimental import pallas as pl
from jax.experimental.pallas import tpu as pltpu
from jax.experimental.pallas import tpu_sc as plsc

_NC = 2
_NS = 16
_NW = _NC * _NS

_B = 16384
_BPW = _B // _NW
_CH = 128
_NCH = _BPW // _CH

_TITLE_D = 128
_TOPIC_D = 64
_DIM = _TITLE_D + 2 * _TOPIC_D


def _sc_gather_body(t_idx, t_tab, out_t, idx_v, rows_t, gsem, wsem):
    wid = lax.axis_index("s") * _NC + lax.axis_index("c")
    base = wid * _BPW

    pltpu.sync_copy(t_idx.at[wid], idx_v)

    def fire(j, slot):
        return pltpu.async_copy(t_tab.at[idx_v.at[j]], rows_t.at[slot], gsem)

    def flush(j, slot):
        off = base + j * _CH
        return pltpu.async_copy(rows_t.at[slot], out_t.at[pl.ds(off, _CH)], wsem)

    pend_g = fire(0, 0)
    pend_w = None
    for j in range(_NCH):
        nxt = None
        if j + 1 < _NCH:
            nxt = fire(j + 1, (j + 1) % 2)
        pend_g.wait()
        if pend_w is not None:
            pend_w.wait()
        pend_w = flush(j, j % 2)
        pend_g = nxt
    pend_w.wait()


def _sc_gather(t_idx, t_tab):
    f = pl.kernel(
        _sc_gather_body,
        out_type=jax.ShapeDtypeStruct((_B, _TITLE_D), jnp.float32),
        mesh=plsc.VectorSubcoreMesh(core_axis_name="c", subcore_axis_name="s",
                                    num_cores=_NC, num_subcores=_NS),
        scratch_types=[
            pltpu.VMEM((_NCH, _CH), jnp.int32),
            pltpu.VMEM((2, _CH, _TITLE_D), jnp.float32),
            pltpu.SemaphoreType.DMA,
            pltpu.SemaphoreType.DMA,
        ],
        name="news_encoder_sc_gather",
    )
    return f(t_idx, t_tab)


_BM = 1024


def _mm_body(t_ref, tp_ref, s_ref, w1_ref, w2_ref, w3_ref, b_ref, o_ref):
    dn = (((1,), (1,)), ((), ()))
    acc = lax.dot_general(t_ref[...], w1_ref[...], dn,
                          preferred_element_type=jnp.float32)
    acc = acc + lax.dot_general(tp_ref[...], w2_ref[...], dn,
                                preferred_element_type=jnp.float32)
    acc = acc + lax.dot_general(s_ref[...], w3_ref[...], dn,
                                preferred_element_type=jnp.float32)
    o_ref[...] = acc + b_ref[...]


def _tc_linear(title, topic, subtopic, W, b):
    w1 = W[:, :_TITLE_D]
    w2 = W[:, _TITLE_D:_TITLE_D + _TOPIC_D]
    w3 = W[:, _TITLE_D + _TOPIC_D:]
    return pl.pallas_call(
        _mm_body,
        grid=(_B // _BM,),
        in_specs=[
            pl.BlockSpec((_BM, _TITLE_D), lambda i: (i, 0)),
            pl.BlockSpec((_BM, _TOPIC_D), lambda i: (i, 0)),
            pl.BlockSpec((_BM, _TOPIC_D), lambda i: (i, 0)),
            pl.BlockSpec((_DIM, _TITLE_D), lambda i: (0, 0)),
            pl.BlockSpec((_DIM, _TOPIC_D), lambda i: (0, 0)),
            pl.BlockSpec((_DIM, _TOPIC_D), lambda i: (0, 0)),
            pl.BlockSpec((1, _DIM), lambda i: (0, 0)),
        ],
        out_specs=pl.BlockSpec((_BM, _DIM), lambda i: (i, 0)),
        out_shape=jax.ShapeDtypeStruct((_B, _DIM), jnp.float32),
    )(title, topic, subtopic, w1, w2, w3, b.reshape(1, _DIM))


def kernel(news_title, news_topic, news_subtopic, title_vectors, topic_table,
           subtopic_table, W, b):
    t_idx = news_title.astype(jnp.int32).reshape(_NW, _NCH, _CH)
    title = _sc_gather(t_idx, title_vectors)
    topic = jnp.take(topic_table, news_topic, axis=0)
    subtopic = jnp.take(subtopic_table, news_subtopic, axis=0)
    return _tc_linear(title, topic, subtopic, W, b)


# speedup vs baseline: 1.9927x; 1.1688x over previous
import functools

import jax
import jax.numpy as jnp
from jax import lax
from jax.experimental import pallas as pl
from jax.experimental.pallas import tpu as pltpu
from jax.experimental.pallas import tpu_sc as plsc

_NC = 2
_NS = 16
_NW = _NC * _NS

_B = 16384
_BPW = _B // _NW
_CH = 128
_NCH = _BPW // _CH

_TITLE_D = 128
_TOPIC_D = 64
_DIM = _TITLE_D + 2 * _TOPIC_D


def _sc_gather_body(t_idx, t_tab, out_t, idx_v, rows_t, gsem, wsem):
    wid = lax.axis_index("s") * _NC + lax.axis_index("c")
    base = wid * _BPW

    pltpu.sync_copy(t_idx.at[wid], idx_v)

    def fire(j, slot):
        return pltpu.async_copy(t_tab.at[idx_v.at[j]], rows_t.at[slot], gsem)

    def flush(j, slot):
        off = base + j * _CH
        return pltpu.async_copy(rows_t.at[slot], out_t.at[pl.ds(off, _CH)], wsem)

    pend_g = fire(0, 0)
    pend_w = None
    for j in range(_NCH):
        nxt = None
        if j + 1 < _NCH:
            nxt = fire(j + 1, (j + 1) % 2)
        pend_g.wait()
        if pend_w is not None:
            pend_w.wait()
        pend_w = flush(j, j % 2)
        pend_g = nxt
    pend_w.wait()


def _sc_gather(t_idx, t_tab):
    f = pl.kernel(
        _sc_gather_body,
        out_type=jax.ShapeDtypeStruct((_B, _TITLE_D), jnp.float32),
        mesh=plsc.VectorSubcoreMesh(core_axis_name="c", subcore_axis_name="s",
                                    num_cores=_NC, num_subcores=_NS),
        scratch_types=[
            pltpu.VMEM((_NCH, _CH), jnp.int32),
            pltpu.VMEM((2, _CH, _TITLE_D), jnp.float32),
            pltpu.SemaphoreType.DMA,
            pltpu.SemaphoreType.DMA,
        ],
        name="news_encoder_sc_gather",
    )
    return f(t_idx, t_tab)


def _sc_gather64_body(tp_idx, s_idx, tp_tab, s_tab, out,
                      idx_v, rows_tp, rows_s, gsem, wsem):
    wid = lax.axis_index("s") * _NC + lax.axis_index("c")
    base = wid * _BPW

    pltpu.sync_copy(tp_idx.at[wid], idx_v.at[0])
    pltpu.sync_copy(s_idx.at[wid], idx_v.at[1])

    def fire(j, slot):
        a = pltpu.async_copy(tp_tab.at[idx_v.at[0, j]], rows_tp.at[slot], gsem)
        b = pltpu.async_copy(s_tab.at[idx_v.at[1, j]], rows_s.at[slot], gsem)
        return a, b

    def flush(j, slot):
        off = base + j * _CH
        a = pltpu.async_copy(rows_tp.at[slot],
                             out.at[pl.ds(off, _CH), pl.ds(0, _TOPIC_D)], wsem)
        b = pltpu.async_copy(rows_s.at[slot],
                             out.at[pl.ds(off, _CH), pl.ds(_TOPIC_D, _TOPIC_D)],
                             wsem)
        return a, b

    pend_g = fire(0, 0)
    pend_w = None
    for j in range(_NCH):
        nxt = None
        if j + 1 < _NCH:
            nxt = fire(j + 1, (j + 1) % 2)
        for h in pend_g:
            h.wait()
        if pend_w is not None:
            for h in pend_w:
                h.wait()
        pend_w = flush(j, j % 2)
        pend_g = nxt
    for h in pend_w:
        h.wait()


def _sc_gather64(tp_idx, s_idx, tp_tab, s_tab):
    f = pl.kernel(
        _sc_gather64_body,
        out_type=jax.ShapeDtypeStruct((_B, 2 * _TOPIC_D), jnp.float32),
        mesh=plsc.VectorSubcoreMesh(core_axis_name="c", subcore_axis_name="s",
                                    num_cores=_NC, num_subcores=_NS),
        scratch_types=[
            pltpu.VMEM((2, _NCH, _CH), jnp.int32),
            pltpu.VMEM((2, _CH, _TOPIC_D), jnp.float32),
            pltpu.VMEM((2, _CH, _TOPIC_D), jnp.float32),
            pltpu.SemaphoreType.DMA,
            pltpu.SemaphoreType.DMA,
        ],
        compiler_params=pltpu.CompilerParams(use_tc_tiling_on_sc=False),
        name="news_encoder_sc_gather64",
    )
    return f(tp_idx, s_idx, tp_tab, s_tab)


_BM = 1024


def _mm_body(t_ref, ts_ref, w1_ref, w23_ref, b_ref, o_ref):
    dn = (((1,), (1,)), ((), ()))
    acc = lax.dot_general(t_ref[...], w1_ref[...], dn,
                          preferred_element_type=jnp.float32)
    acc = acc + lax.dot_general(ts_ref[...], w23_ref[...], dn,
                                preferred_element_type=jnp.float32)
    o_ref[...] = acc + b_ref[...]


def _tc_linear(title, topic_sub, W, b):
    w1 = W[:, :_TITLE_D]
    w23 = W[:, _TITLE_D:]
    return pl.pallas_call(
        _mm_body,
        grid=(_B // _BM,),
        in_specs=[
            pl.BlockSpec((_BM, _TITLE_D), lambda i: (i, 0)),
            pl.BlockSpec((_BM, 2 * _TOPIC_D), lambda i: (i, 0)),
            pl.BlockSpec((_DIM, _TITLE_D), lambda i: (0, 0)),
            pl.BlockSpec((_DIM, 2 * _TOPIC_D), lambda i: (0, 0)),
            pl.BlockSpec((1, _DIM), lambda i: (0, 0)),
        ],
        out_specs=pl.BlockSpec((_BM, _DIM), lambda i: (i, 0)),
        out_shape=jax.ShapeDtypeStruct((_B, _DIM), jnp.float32),
    )(title, topic_sub, w1, w23, b.reshape(1, _DIM))


def kernel(news_title, news_topic, news_subtopic, title_vectors, topic_table,
           subtopic_table, W, b):
    t_idx = news_title.astype(jnp.int32).reshape(_NW, _NCH, _CH)
    tp_idx = news_topic.astype(jnp.int32).reshape(_NW, _NCH, _CH)
    s_idx = news_subtopic.astype(jnp.int32).reshape(_NW, _NCH, _CH)
    title = _sc_gather(t_idx, title_vectors)
    topic_sub = _sc_gather64(tp_idx, s_idx, topic_table, subtopic_table)
    return _tc_linear(title, topic_sub, W, b)
